# Initial kernel scaffold; baseline (speedup 1.0000x reference)
import jax
import jax.numpy as jnp
from jax.experimental import pallas as pl


def kernel(pbboxs, plabels, gbboxs, glabels, ancs):
    raise NotImplementedError("write your pallas kernel here")


# speedup vs baseline: 10.3347x; 10.3347x over previous
import jax
import jax.numpy as jnp
from jax.experimental import pallas as pl
from jax.experimental.pallas import tpu as pltpu

_B = 16
_N = 100000
_R = 200
_L = 500
_RB = 8
_STEPS = _R // _RB
_SCALE_XY = 10.0
_SCALE_WH = 5.0
_NEG_RATIO = 3.0
_EPS = 1.1920928955078125e-07


def _smooth_l1(d):
    ad = jnp.abs(d)
    return jnp.where(ad < 1.0, 0.5 * d * d, ad - 0.5)


def _rowsum(x):
    return jnp.sum(jnp.sum(x, axis=2), axis=1)[:, None]


def _loss_kernel(px, py, pw, ph, gx, gy, gw, gh, plab, glab,
                 ax, ay, aw, ah, out_ref,
                 bits_ref, ll_ref, pos_ref, bb_ref, tot_ref, pbce_ref,
                 sr_ref):
    i = pl.program_id(0)

    @pl.when(i == 0)
    def _init():
        pos_ref[...] = jnp.zeros_like(pos_ref)
        bb_ref[...] = jnp.zeros_like(bb_ref)
        tot_ref[...] = jnp.zeros_like(tot_ref)
        pbce_ref[...] = jnp.zeros_like(pbce_ref)
        sr_ref[...] = jnp.zeros_like(sr_ref)

    m = (glab[...] > 0.0).astype(jnp.float32)

    dx = px[...] - _SCALE_XY * (gx[...] - ax[...]) / aw[...]
    dy = py[...] - _SCALE_XY * (gy[...] - ay[...]) / ah[...]
    dw = pw[...] - _SCALE_WH * jnp.log(gw[...] / aw[...])
    dh = ph[...] - _SCALE_WH * jnp.log(gh[...] / ah[...])
    sl1 = _smooth_l1(dx) + _smooth_l1(dy) + _smooth_l1(dw) + _smooth_l1(dh)

    x = plab[...]
    bce = (jnp.maximum(x, 0.0) - x * glab[...]
           + jnp.log(1.0 + jnp.exp(-jnp.abs(x))))
    lneg = jnp.where(m > 0.0, 0.0, bce)

    sl = pl.ds(i * _RB, _RB)
    bits_ref[:, sl, :] = jax.lax.bitcast_convert_type(lneg, jnp.int32)
    ll_ref[:, sl, :] = bce

    pos_ref[...] = pos_ref[...] + _rowsum(m)
    bb_ref[...] = bb_ref[...] + _rowsum(m * sl1)
    tot_ref[...] = tot_ref[...] + _rowsum(bce)
    pbce_ref[...] = pbce_ref[...] + _rowsum(m * bce)

    @pl.when(i == _STEPS - 1)
    def _finish():
        pos = pos_ref[...]
        k = jnp.minimum(_NEG_RATIO * pos, float(_N))
        need = jnp.any((pos > 0.0) & (k < float(_N)))

        @pl.when(need)
        def _search():
            bits = bits_ref[...]
            ll = ll_ref[...]

            def cnt3(mask):
                return jnp.sum(jnp.sum(mask.astype(jnp.float32), axis=2),
                               axis=1)[:, None]

            def vstep(sh, lohi):
                lo, hi = lohi
                mid = lo + jax.lax.shift_right_logical(hi - lo + 1, 1)
                ok = cnt3(bits >= mid[:, :, None]) >= k
                return jnp.where(ok, mid, lo), jnp.where(ok, hi, mid - 1)

            lo0 = jnp.zeros((_B, 1), jnp.int32)
            hi0 = jnp.full((_B, 1), jnp.int32(0x7FFFFFFF))
            t, _ = jax.lax.fori_loop(0, 31, vstep, (lo0, hi0))

            t3 = t[:, :, None]
            gt = bits > t3
            sum_gt = jnp.sum(jnp.sum(jnp.where(gt, ll, 0.0), axis=2),
                             axis=1)[:, None]
            r = k - cnt3(gt)
            eq = bits == t3
            idx = (jax.lax.broadcasted_iota(jnp.int32, (_B, _R, _L), 1) * _L
                   + jax.lax.broadcasted_iota(jnp.int32, (_B, _R, _L), 2))

            def istep(sh, lohi):
                lo, hi = lohi
                mid = jax.lax.shift_right_logical(lo + hi, 1)
                ok = cnt3(eq & (idx < mid[:, :, None])) >= r
                return jnp.where(ok, lo, mid + 1), jnp.where(ok, mid, hi)

            ilo = jnp.zeros((_B, 1), jnp.int32)
            ihi = jnp.full((_B, 1), jnp.int32(_N))
            mth, _ = jax.lax.fori_loop(0, 18, istep, (ilo, ihi))

            tie = jnp.sum(jnp.sum(
                jnp.where(eq & (idx < mth[:, :, None]), ll, 0.0),
                axis=2), axis=1)[:, None]
            sr_ref[...] = sum_gt + tie

        neg = jnp.where(k >= float(_N), tot_ref[...], sr_ref[...])

        num_mask = (pos > 0.0).astype(jnp.float32)
        pos_f = jnp.maximum(pos, _EPS)
        w = num_mask / pos_f
        lb_s = jnp.sum(bb_ref[...] * w) / _B
        ll_s = jnp.sum((pbce_ref[...] + neg) * w) / _B
        total = (lb_s + ll_s) * (jnp.sum(w) / _B)
        lane = jax.lax.broadcasted_iota(jnp.int32, (1, 128), 1)
        vals = jnp.where(lane == 0, total,
                         jnp.where(lane == 1, lb_s,
                                   jnp.where(lane == 2, ll_s, 0.0)))
        out_ref[...] = vals


@jax.jit
def kernel(pbboxs, plabels, gbboxs, glabels, ancs):
    def v3(a):
        return a.reshape(_B, _R, _L)

    pc = [v3(pbboxs[:, :, j]) for j in range(4)]
    gc = [v3(gbboxs[:, :, j]) for j in range(4)]
    ac = [ancs[:, j].reshape(1, _R, _L) for j in range(4)]

    row_spec = pl.BlockSpec((_B, _RB, _L), lambda i: (0, i, 0))
    anc_spec = pl.BlockSpec((1, _RB, _L), lambda i: (0, i, 0))

    out = pl.pallas_call(
        _loss_kernel,
        grid=(_STEPS,),
        in_specs=[row_spec] * 8 + [row_spec] * 2 + [anc_spec] * 4,
        out_specs=pl.BlockSpec((1, 128), lambda i: (0, 0)),
        out_shape=jax.ShapeDtypeStruct((1, 128), jnp.float32),
        scratch_shapes=[
            pltpu.VMEM((_B, _R, _L), jnp.int32),
            pltpu.VMEM((_B, _R, _L), jnp.float32),
            pltpu.VMEM((_B, 1), jnp.float32),
            pltpu.VMEM((_B, 1), jnp.float32),
            pltpu.VMEM((_B, 1), jnp.float32),
            pltpu.VMEM((_B, 1), jnp.float32),
            pltpu.VMEM((_B, 1), jnp.float32),
        ],
    )(*pc, *gc, v3(plabels), v3(glabels), *ac)
    return (out[0, 0], out[0, 1], out[0, 2])
